# baseline (speedup 1.0000x reference)
import functools

import jax
import jax.numpy as jnp
from jax import lax
from jax.experimental import pallas as pl
from jax.experimental.pallas import tpu as pltpu
from jax.experimental.pallas import tpu_sc as plsc

LANES = 16
NC = 2
NS = 16
NW = NC * NS

D = 64
ND = D // LANES
SG = 16
LC = 40
LO = 8

_GDN = lax.GatherDimensionNumbers(
    offset_dims=(), collapsed_slice_dims=(0,), start_index_map=(0,))


def _lane_bcast(v, lane):
    idx = jnp.full((LANES, 1), lane, dtype=jnp.int32)
    return lax.gather(v, idx, _GDN, (1,),
                      mode=lax.GatherScatterMode.PROMISE_IN_BOUNDS)


def _rsqrt_vec(x):
    i = lax.bitcast_convert_type(x, jnp.int32)
    i = jnp.int32(0x5F3759DF) - lax.shift_right_arithmetic(i, jnp.int32(1))
    y = lax.bitcast_convert_type(i, jnp.float32)
    for _ in range(2):
        y = y * (1.5 - 0.5 * x * y * y)
    return y


def _make_kernel(B, L, V):
    assert B % (NW * 128) == 0 and L % LC == 0
    ncell = (B // NW // SG) * (L // LC)
    ngrp_l = L // LC

    mesh = plsc.VectorSubcoreMesh(core_axis_name="c", subcore_axis_name="s")

    scratch = (
        [pltpu.VMEM((SG, LC, D), jnp.float32) for _ in range(2)]
        + [pltpu.VMEM((LO, 8, 8, SG), jnp.float32) for _ in range(2)]
        + [pltpu.VMEM((SG, LC), jnp.int32) for _ in range(2)]
        + [pltpu.VMEM((SG, LC), jnp.int32) for _ in range(2)]
        + [
            pltpu.VMEM((L, D), jnp.float32),
            pltpu.VMEM((2, D), jnp.float32),
            pltpu.VMEM((D,), jnp.float32),
            pltpu.VMEM((D,), jnp.float32),
            pltpu.VMEM((D,), jnp.float32),
        ]
        + [pltpu.SemaphoreType.DMA for _ in range(6)]
    )

    @functools.partial(
        pl.kernel,
        out_type=jax.ShapeDtypeStruct((L, D // 8, NW * 128 // 128, 8, 128),
                                      jnp.float32),
        mesh=mesh,
        compiler_params=pltpu.CompilerParams(
            needs_layout_passes=False, use_tc_tiling_on_sc=False),
        scratch_types=scratch,
    )
    def k(x_hbm, seg_hbm, tok_hbm, pos_hbm, segt_hbm, gam_hbm, bet_hbm,
          out_hbm, *refs):
        gbuf = refs[0:2]
        obuf = refs[2:4]
        xids = refs[4:6]
        sgids = refs[6:8]
        posb, segt_v, dd_v, gm_v, bt_v = refs[8:13]
        sems = refs[13:]
        sem_g = sems[0:2]
        sem_i = sems[2:4]
        sem_o = sems[4:6]

        wid = lax.axis_index("s") * NC + lax.axis_index("c")
        b00 = wid * 128

        pltpu.sync_copy(pos_hbm.at[pl.ds(0, L)], posb)
        pltpu.sync_copy(segt_hbm, segt_v)
        pltpu.sync_copy(gam_hbm, gm_v)
        pltpu.sync_copy(bet_hbm, bt_v)

        sl = [pl.ds(kk * LANES, LANES) for kk in range(ND)]

        def _fold_seg0(r, carry):
            for kk in range(ND):
                posb[r, sl[kk]] = posb[r, sl[kk]] + segt_v[0, sl[kk]]
            return carry
        lax.fori_loop(0, L, _fold_seg0, 0)
        for kk in range(ND):
            dd_v[sl[kk]] = segt_v[1, sl[kk]] - segt_v[0, sl[kk]]

        lanev = lax.iota(jnp.int32, LANES)
        dtv = [2 * kk + lax.shift_right_logical(lanev, jnp.int32(3))
               for kk in range(ND)]
        dsv = lanev & jnp.int32(7)

        def cell_gb(n):
            g = n // ngrp_l
            c = n % ngrp_l
            return b00 + g * SG, c * LC, g

        def stage(n, b, issue):
            b0, l0, _ = cell_gb(n)
            src_x = x_hbm.at[pl.ds(b0, SG), pl.ds(l0, LC)]
            src_s = seg_hbm.at[pl.ds(b0, SG), pl.ds(l0, LC)]
            if issue:
                pltpu.async_copy(src_x, xids[b], sem_i[b])
                pltpu.async_copy(src_s, sgids[b], sem_i[b])
            else:
                pltpu.make_async_copy(src_x, xids[b], sem_i[b]).wait()
                pltpu.make_async_copy(src_s, sgids[b], sem_i[b]).wait()

        def gathers(b, issue):
            for q in range(SG):
                src = tok_hbm.at[xids[b].at[q, :]]
                dst = gbuf[b].at[q]
                if issue:
                    pltpu.async_copy(src, dst, sem_g[b])
                else:
                    pltpu.make_async_copy(src, dst, sem_g[b]).wait()

        def out_dma(n, j, bo, issue):
            _, l0, g = cell_gb(n)
            dst = out_hbm.at[pl.ds(l0 + j * LO, LO), :, wid, :,
                             pl.ds(g * SG, SG)]
            if issue:
                pltpu.async_copy(obuf[bo], dst, sem_o[bo])
            else:
                pltpu.make_async_copy(obuf[bo], dst, sem_o[bo]).wait()

        def _token(gb, ob, sg, q, lc, gl, lq, dd, gm, bt):
            tok = [gb[q, lc, sl[kk]] for kk in range(ND)]
            pos = [posb[gl, sl[kk]] for kk in range(ND)]
            qv = jnp.full((LANES,), q, jnp.int32)
            lv = jnp.full((LANES,), lc, jnp.int32)
            sb = plsc.load_gather(sg, [qv, lv]).astype(jnp.float32)
            e = [tok[kk] + pos[kk] + sb * dd[kk] for kk in range(ND)]
            tot = (e[0] + e[1]) + (e[2] + e[3])
            mean = _lane_bcast(plsc.cumsum(tot), LANES - 1) * (1.0 / D)
            c = [e[kk] - mean for kk in range(ND)]
            sq = (c[0] * c[0] + c[1] * c[1]) + (c[2] * c[2] + c[3] * c[3])
            var = _lane_bcast(plsc.cumsum(sq), LANES - 1) * (1.0 / D)
            rv = _rsqrt_vec(var + 1e-5)
            lqv = jnp.full((LANES,), lq, jnp.int32)
            for kk in range(ND):
                val = c[kk] * (rv * gm[kk]) + bt[kk]
                plsc.store_scatter(ob, [lqv, dtv[kk], dsv, qv], val)

        def compute_tile(n, b, j, bo):
            _, l0, _ = cell_gb(n)
            dd = [dd_v[sl[kk]] for kk in range(ND)]
            gm = [gm_v[sl[kk]] for kk in range(ND)]
            bt = [bt_v[sl[kk]] for kk in range(ND)]

            def _lbody(lq, carry):
                lc = j * LO + lq
                gl = l0 + lc

                def _qbody(q, carry2):
                    _token(gbuf[b], obuf[bo], sgids[b], q, lc, gl, lq,
                           dd, gm, bt)
                    return carry2
                lax.fori_loop(0, SG, _qbody, 0, unroll=4)
                return carry
            lax.fori_loop(0, LO, _lbody, 0)

        pltpu.sync_copy(x_hbm.at[pl.ds(b00, SG), pl.ds(0, LC)], xids[0])
        pltpu.sync_copy(seg_hbm.at[pl.ds(b00, SG), pl.ds(0, LC)], sgids[0])
        stage(1, 1, True)
        gathers(0, True)

        def _iter(p, carry):
            for k2 in range(2):
                n = p * 2 + k2
                b = k2
                bn = 1 - k2
                gathers(b, False)
                if k2 == 0:
                    stage(n + 1, bn, False)
                    gathers(bn, True)
                else:
                    @pl.when(p <= ncell // 2 - 2)
                    def _g():
                        stage(n + 1, bn, False)
                        gathers(bn, True)
                for j in range(ngrp_l):
                    bo = (k2 + j) % 2
                    if k2 == 0 and j < 2:
                        @pl.when(p >= 1)
                        def _w(n=n, j=j, bo=bo):
                            out_dma(n, j, bo, False)
                    else:
                        out_dma(n, j, bo, False)
                    compute_tile(n, b, j, bo)
                    out_dma(n, j, bo, True)
                @pl.when(p <= ncell // 2 - 2)
                def _s():
                    stage(n + 2, b, True)
            return carry
        lax.fori_loop(0, ncell // 2, _iter, 0)

        out_dma(ncell - 1, ngrp_l - 2, (ncell - 1 + ngrp_l - 2) % 2, False)
        out_dma(ncell - 1, ngrp_l - 1, (ncell - 1 + ngrp_l - 1) % 2, False)

    return k


def kernel(x, seg, token_table, pos_table, seg_table, gamma, beta):
    B, L = x.shape
    V, d = token_table.shape
    k = _make_kernel(B, L, V)
    out5 = k(x.astype(jnp.int32), seg.astype(jnp.int32),
             token_table, pos_table, seg_table, gamma, beta)
    return out5.transpose(2, 4, 0, 1, 3).reshape(B, L, d)


# speedup vs baseline: 1.0801x; 1.0801x over previous
import functools

import jax
import jax.numpy as jnp
from jax import lax
from jax.experimental import pallas as pl
from jax.experimental.pallas import tpu as pltpu
from jax.experimental.pallas import tpu_sc as plsc

LANES = 16
NC = 2
NS = 16
NW = NC * NS

D = 64
ND = D // LANES
SG = 16
LC = 40
LO = 8

_GDN = lax.GatherDimensionNumbers(
    offset_dims=(), collapsed_slice_dims=(0,), start_index_map=(0,))


def _lane_bcast(v, lane):
    idx = jnp.full((LANES, 1), lane, dtype=jnp.int32)
    return lax.gather(v, idx, _GDN, (1,),
                      mode=lax.GatherScatterMode.PROMISE_IN_BOUNDS)


def _rsqrt_vec(x):
    i = lax.bitcast_convert_type(x, jnp.int32)
    i = jnp.int32(0x5F3759DF) - lax.shift_right_arithmetic(i, jnp.int32(1))
    y = lax.bitcast_convert_type(i, jnp.float32)
    for _ in range(2):
        y = y * (1.5 - 0.5 * x * y * y)
    return y


def _make_kernel(B, L, V):
    assert B % (NW * 128) == 0 and L % LC == 0
    ncell = (B // NW // SG) * (L // LC)
    ngrp_l = L // LC

    mesh = plsc.VectorSubcoreMesh(core_axis_name="c", subcore_axis_name="s")

    scratch = (
        [pltpu.VMEM((SG, LC, D), jnp.float32) for _ in range(2)]
        + [pltpu.VMEM((LO, 8, 8, SG + 1), jnp.float32) for _ in range(2)]
        + [pltpu.VMEM((SG, LC), jnp.int32) for _ in range(2)]
        + [pltpu.VMEM((SG, LC), jnp.int32) for _ in range(2)]
        + [
            pltpu.VMEM((L, D), jnp.float32),
            pltpu.VMEM((2, D), jnp.float32),
            pltpu.VMEM((D,), jnp.float32),
            pltpu.VMEM((D,), jnp.float32),
            pltpu.VMEM((D,), jnp.float32),
        ]
        + [pltpu.SemaphoreType.DMA for _ in range(6)]
    )

    @functools.partial(
        pl.kernel,
        out_type=jax.ShapeDtypeStruct((L, D // 8, NW * 128 // 128, 8, 128),
                                      jnp.float32),
        mesh=mesh,
        compiler_params=pltpu.CompilerParams(
            needs_layout_passes=False, use_tc_tiling_on_sc=False),
        scratch_types=scratch,
    )
    def k(x_hbm, seg_hbm, tok_hbm, pos_hbm, segt_hbm, gam_hbm, bet_hbm,
          out_hbm, *refs):
        gbuf = refs[0:2]
        obuf = refs[2:4]
        xids = refs[4:6]
        sgids = refs[6:8]
        posb, segt_v, dd_v, gm_v, bt_v = refs[8:13]
        sems = refs[13:]
        sem_g = sems[0:2]
        sem_i = sems[2:4]
        sem_o = sems[4:6]

        wid = lax.axis_index("s") * NC + lax.axis_index("c")
        b00 = wid * 128

        pltpu.sync_copy(pos_hbm.at[pl.ds(0, L)], posb)
        pltpu.sync_copy(segt_hbm, segt_v)
        pltpu.sync_copy(gam_hbm, gm_v)
        pltpu.sync_copy(bet_hbm, bt_v)

        sl = [pl.ds(kk * LANES, LANES) for kk in range(ND)]

        def _fold_seg0(r, carry):
            for kk in range(ND):
                posb[r, sl[kk]] = posb[r, sl[kk]] + segt_v[0, sl[kk]]
            return carry
        lax.fori_loop(0, L, _fold_seg0, 0)
        for kk in range(ND):
            dd_v[sl[kk]] = segt_v[1, sl[kk]] - segt_v[0, sl[kk]]

        lanev = lax.iota(jnp.int32, LANES)
        dtv = [2 * kk + lax.shift_right_logical(lanev, jnp.int32(3))
               for kk in range(ND)]
        dsv = lanev & jnp.int32(7)

        def cell_gb(n):
            g = n // ngrp_l
            c = n % ngrp_l
            return b00 + g * SG, c * LC, g

        def stage(n, b, issue):
            b0, l0, _ = cell_gb(n)
            src_x = x_hbm.at[pl.ds(b0, SG), pl.ds(l0, LC)]
            src_s = seg_hbm.at[pl.ds(b0, SG), pl.ds(l0, LC)]
            if issue:
                pltpu.async_copy(src_x, xids[b], sem_i[b])
                pltpu.async_copy(src_s, sgids[b], sem_i[b])
            else:
                pltpu.make_async_copy(src_x, xids[b], sem_i[b]).wait()
                pltpu.make_async_copy(src_s, sgids[b], sem_i[b]).wait()

        def gathers(b, issue):
            for q in range(SG):
                src = tok_hbm.at[xids[b].at[q, :]]
                dst = gbuf[b].at[q]
                if issue:
                    pltpu.async_copy(src, dst, sem_g[b])
                else:
                    pltpu.make_async_copy(src, dst, sem_g[b]).wait()

        def out_dma(n, j, bo, issue):
            _, l0, g = cell_gb(n)
            dst = out_hbm.at[pl.ds(l0 + j * LO, LO), :, wid, :,
                             pl.ds(g * SG, SG)]
            src = obuf[bo].at[:, :, :, pl.ds(0, SG)]
            if issue:
                pltpu.async_copy(src, dst, sem_o[bo])
            else:
                pltpu.make_async_copy(src, dst, sem_o[bo]).wait()

        def _token(gb, ob, sg, q, lc, gl, lq, dd, gm, bt):
            tok = [gb[q, lc, sl[kk]] for kk in range(ND)]
            pos = [posb[gl, sl[kk]] for kk in range(ND)]
            qv = jnp.full((LANES,), q, jnp.int32)
            lv = jnp.full((LANES,), lc, jnp.int32)
            sb = plsc.load_gather(sg, [qv, lv]).astype(jnp.float32)
            e = [tok[kk] + pos[kk] + sb * dd[kk] for kk in range(ND)]
            tot = (e[0] + e[1]) + (e[2] + e[3])
            mean = _lane_bcast(plsc.cumsum(tot), LANES - 1) * (1.0 / D)
            c = [e[kk] - mean for kk in range(ND)]
            sq = (c[0] * c[0] + c[1] * c[1]) + (c[2] * c[2] + c[3] * c[3])
            var = _lane_bcast(plsc.cumsum(sq), LANES - 1) * (1.0 / D)
            rv = _rsqrt_vec(var + 1e-5)
            lqv = jnp.full((LANES,), lq, jnp.int32)
            for kk in range(ND):
                val = c[kk] * (rv * gm[kk]) + bt[kk]
                plsc.store_scatter(ob, [lqv, dtv[kk], dsv, qv], val)

        def compute_tile(n, b, j, bo):
            _, l0, _ = cell_gb(n)
            dd = [dd_v[sl[kk]] for kk in range(ND)]
            gm = [gm_v[sl[kk]] for kk in range(ND)]
            bt = [bt_v[sl[kk]] for kk in range(ND)]

            def _lbody(lq, carry):
                lc = j * LO + lq
                gl = l0 + lc

                def _qbody(q, carry2):
                    _token(gbuf[b], obuf[bo], sgids[b], q, lc, gl, lq,
                           dd, gm, bt)
                    return carry2
                lax.fori_loop(0, SG, _qbody, 0, unroll=4)
                return carry
            lax.fori_loop(0, LO, _lbody, 0)

        pltpu.sync_copy(x_hbm.at[pl.ds(b00, SG), pl.ds(0, LC)], xids[0])
        pltpu.sync_copy(seg_hbm.at[pl.ds(b00, SG), pl.ds(0, LC)], sgids[0])
        stage(1, 1, True)
        gathers(0, True)

        def _iter(p, carry):
            for k2 in range(2):
                n = p * 2 + k2
                b = k2
                bn = 1 - k2
                gathers(b, False)
                if k2 == 0:
                    stage(n + 1, bn, False)
                    gathers(bn, True)
                else:
                    @pl.when(p <= ncell // 2 - 2)
                    def _g():
                        stage(n + 1, bn, False)
                        gathers(bn, True)
                for j in range(ngrp_l):
                    bo = (k2 + j) % 2
                    if k2 == 0 and j < 2:
                        @pl.when(p >= 1)
                        def _w(n=n, j=j, bo=bo):
                            out_dma(n, j, bo, False)
                    else:
                        out_dma(n, j, bo, False)
                    compute_tile(n, b, j, bo)
                    out_dma(n, j, bo, True)
                @pl.when(p <= ncell // 2 - 2)
                def _s():
                    stage(n + 2, b, True)
            return carry
        lax.fori_loop(0, ncell // 2, _iter, 0)

        out_dma(ncell - 1, ngrp_l - 2, (ncell - 1 + ngrp_l - 2) % 2, False)
        out_dma(ncell - 1, ngrp_l - 1, (ncell - 1 + ngrp_l - 1) % 2, False)

    return k


def kernel(x, seg, token_table, pos_table, seg_table, gamma, beta):
    B, L = x.shape
    V, d = token_table.shape
    k = _make_kernel(B, L, V)
    out5 = k(x.astype(jnp.int32), seg.astype(jnp.int32),
             token_table, pos_table, seg_table, gamma, beta)
    return out5.transpose(2, 4, 0, 1, 3).reshape(B, L, d)


# speedup vs baseline: 1.0981x; 1.0167x over previous
import functools

import jax
import jax.numpy as jnp
from jax import lax
from jax.experimental import pallas as pl
from jax.experimental.pallas import tpu as pltpu
from jax.experimental.pallas import tpu_sc as plsc

LANES = 16
NC = 2
NS = 16
NW = NC * NS

D = 64
ND = D // LANES
SG = 16
LC = 40
LO = 8

_GDN = lax.GatherDimensionNumbers(
    offset_dims=(), collapsed_slice_dims=(0,), start_index_map=(0,))


def _lane_bcast(v, lane):
    idx = jnp.full((LANES, 1), lane, dtype=jnp.int32)
    return lax.gather(v, idx, _GDN, (1,),
                      mode=lax.GatherScatterMode.PROMISE_IN_BOUNDS)


def _rsqrt_vec(x):
    i = lax.bitcast_convert_type(x, jnp.int32)
    i = jnp.int32(0x5F3759DF) - lax.shift_right_arithmetic(i, jnp.int32(1))
    y = lax.bitcast_convert_type(i, jnp.float32)
    for _ in range(2):
        y = y * (1.5 - 0.5 * x * y * y)
    return y


def _make_kernel(B, L, V):
    assert B % (NW * 128) == 0 and L % LC == 0
    ncell = (B // NW // SG) * (L // LC)
    ngrp_l = L // LC

    mesh = plsc.VectorSubcoreMesh(core_axis_name="c", subcore_axis_name="s")

    scratch = (
        [pltpu.VMEM((SG, LC, D), jnp.float32) for _ in range(2)]
        + [pltpu.VMEM((LO, 8, 8, SG + 1), jnp.float32) for _ in range(2)]
        + [pltpu.VMEM((SG, LC), jnp.int32) for _ in range(2)]
        + [pltpu.VMEM((SG, LC + 8), jnp.int32) for _ in range(2)]
        + [
            pltpu.VMEM((L, D), jnp.float32),
            pltpu.VMEM((2, D), jnp.float32),
            pltpu.VMEM((D,), jnp.float32),
            pltpu.VMEM((D,), jnp.float32),
            pltpu.VMEM((D,), jnp.float32),
        ]
        + [pltpu.SemaphoreType.DMA for _ in range(6)]
    )

    @functools.partial(
        pl.kernel,
        out_type=jax.ShapeDtypeStruct((L, D // 8, NW * 128 // 128, 8, 128),
                                      jnp.float32),
        mesh=mesh,
        compiler_params=pltpu.CompilerParams(
            needs_layout_passes=False, use_tc_tiling_on_sc=False),
        scratch_types=scratch,
    )
    def k(x_hbm, seg_hbm, tok_hbm, pos_hbm, segt_hbm, gam_hbm, bet_hbm,
          out_hbm, *refs):
        gbuf = refs[0:2]
        obuf = refs[2:4]
        xids = refs[4:6]
        sgids = refs[6:8]
        posb, segt_v, dd_v, gm_v, bt_v = refs[8:13]
        sems = refs[13:]
        sem_g = sems[0:2]
        sem_i = sems[2:4]
        sem_o = sems[4:6]

        wid = lax.axis_index("s") * NC + lax.axis_index("c")
        b00 = wid * 128

        pltpu.sync_copy(pos_hbm.at[pl.ds(0, L)], posb)
        pltpu.sync_copy(segt_hbm, segt_v)
        pltpu.sync_copy(gam_hbm, gm_v)
        pltpu.sync_copy(bet_hbm, bt_v)

        sl = [pl.ds(kk * LANES, LANES) for kk in range(ND)]

        def _fold_seg0(r, carry):
            for kk in range(ND):
                posb[r, sl[kk]] = posb[r, sl[kk]] + segt_v[0, sl[kk]]
            return carry
        lax.fori_loop(0, L, _fold_seg0, 0)
        for kk in range(ND):
            dd_v[sl[kk]] = segt_v[1, sl[kk]] - segt_v[0, sl[kk]]

        lanev = lax.iota(jnp.int32, LANES)
        dtv = [2 * kk + lax.shift_right_logical(lanev, jnp.int32(3))
               for kk in range(ND)]
        dsv = lanev & jnp.int32(7)

        def cell_gb(n):
            g = n // ngrp_l
            c = n % ngrp_l
            return b00 + g * SG, c * LC, g

        def stage(n, b, issue):
            b0, l0, _ = cell_gb(n)
            src_x = x_hbm.at[pl.ds(b0, SG), pl.ds(l0, LC)]
            src_s = seg_hbm.at[pl.ds(b0, SG), pl.ds(l0, LC)]
            dst_s = sgids[b].at[pl.ds(0, SG), pl.ds(0, LC)]
            if issue:
                pltpu.async_copy(src_x, xids[b], sem_i[b])
                pltpu.async_copy(src_s, dst_s, sem_i[b])
            else:
                pltpu.make_async_copy(src_x, xids[b], sem_i[b]).wait()
                pltpu.make_async_copy(src_s, dst_s, sem_i[b]).wait()

        def gathers(b, issue):
            for q in range(SG):
                src = tok_hbm.at[xids[b].at[q, :]]
                dst = gbuf[b].at[q]
                if issue:
                    pltpu.async_copy(src, dst, sem_g[b])
                else:
                    pltpu.make_async_copy(src, dst, sem_g[b]).wait()

        def out_dma(n, j, bo, issue):
            _, l0, g = cell_gb(n)
            dst = out_hbm.at[pl.ds(l0 + j * LO, LO), :, wid, :,
                             pl.ds(g * SG, SG)]
            src = obuf[bo].at[:, :, :, pl.ds(0, SG)]
            if issue:
                pltpu.async_copy(src, dst, sem_o[bo])
            else:
                pltpu.make_async_copy(src, dst, sem_o[bo]).wait()

        def _token(gb, ob, q, qv, lc, gl, li, sf16, dd, gm, bt):
            tok = [gb[q, lc, sl[kk]] for kk in range(ND)]
            pos = [posb[gl, sl[kk]] for kk in range(ND)]
            sb = _lane_bcast(sf16, li)
            e = [tok[kk] + pos[kk] + sb * dd[kk] for kk in range(ND)]
            tot = (e[0] + e[1]) + (e[2] + e[3])
            mean = _lane_bcast(plsc.cumsum(tot), LANES - 1) * (1.0 / D)
            c = [e[kk] - mean for kk in range(ND)]
            sq = (c[0] * c[0] + c[1] * c[1]) + (c[2] * c[2] + c[3] * c[3])
            var = _lane_bcast(plsc.cumsum(sq), LANES - 1) * (1.0 / D)
            rv = _rsqrt_vec(var + 1e-5)
            lqv = jnp.full((LANES,), li, jnp.int32)
            for kk in range(ND):
                val = c[kk] * (rv * gm[kk]) + bt[kk]
                plsc.store_scatter(ob, [lqv, dtv[kk], dsv, qv], val)

        def compute_tile(n, b, j, bo):
            _, l0, _ = cell_gb(n)
            dd = [dd_v[sl[kk]] for kk in range(ND)]
            gm = [gm_v[sl[kk]] for kk in range(ND)]
            bt = [bt_v[sl[kk]] for kk in range(ND)]

            def _qbody(q, carry):
                qv = jnp.full((LANES,), q, jnp.int32)
                sf16 = sgids[b][q, pl.ds(j * LO, 16)].astype(jnp.float32)
                for li in range(LO):
                    lc = j * LO + li
                    _token(gbuf[b], obuf[bo], q, qv, lc, l0 + lc, li,
                           sf16, dd, gm, bt)
                return carry
            lax.fori_loop(0, SG, _qbody, 0)

        pltpu.sync_copy(x_hbm.at[pl.ds(b00, SG), pl.ds(0, LC)], xids[0])
        pltpu.sync_copy(seg_hbm.at[pl.ds(b00, SG), pl.ds(0, LC)],
                        sgids[0].at[pl.ds(0, SG), pl.ds(0, LC)])
        stage(1, 1, True)
        gathers(0, True)

        def _iter(p, carry):
            for k2 in range(2):
                n = p * 2 + k2
                b = k2
                bn = 1 - k2
                gathers(b, False)
                if k2 == 0:
                    stage(n + 1, bn, False)
                    gathers(bn, True)
                else:
                    @pl.when(p <= ncell // 2 - 2)
                    def _g():
                        stage(n + 1, bn, False)
                        gathers(bn, True)
                for j in range(ngrp_l):
                    bo = (k2 + j) % 2
                    if k2 == 0 and j < 2:
                        @pl.when(p >= 1)
                        def _w(n=n, j=j, bo=bo):
                            out_dma(n, j, bo, False)
                    else:
                        out_dma(n, j, bo, False)
                    compute_tile(n, b, j, bo)
                    out_dma(n, j, bo, True)
                @pl.when(p <= ncell // 2 - 2)
                def _s():
                    stage(n + 2, b, True)
            return carry
        lax.fori_loop(0, ncell // 2, _iter, 0)

        out_dma(ncell - 1, ngrp_l - 2, (ncell - 1 + ngrp_l - 2) % 2, False)
        out_dma(ncell - 1, ngrp_l - 1, (ncell - 1 + ngrp_l - 1) % 2, False)

    return k


def kernel(x, seg, token_table, pos_table, seg_table, gamma, beta):
    B, L = x.shape
    V, d = token_table.shape
    k = _make_kernel(B, L, V)
    out5 = k(x.astype(jnp.int32), seg.astype(jnp.int32),
             token_table, pos_table, seg_table, gamma, beta)
    return out5.transpose(2, 4, 0, 1, 3).reshape(B, L, d)


# speedup vs baseline: 1.1900x; 1.0836x over previous
import functools

import jax
import jax.numpy as jnp
from jax import lax
from jax.experimental import pallas as pl
from jax.experimental.pallas import tpu as pltpu
from jax.experimental.pallas import tpu_sc as plsc

LANES = 16
NC = 2
NS = 16
NW = NC * NS

D = 64
ND = D // LANES
SG = 16
LC = 40
LO = 8

_GDN = lax.GatherDimensionNumbers(
    offset_dims=(), collapsed_slice_dims=(0,), start_index_map=(0,))


def _lane_bcast(v, lane):
    idx = jnp.full((LANES, 1), lane, dtype=jnp.int32)
    return lax.gather(v, idx, _GDN, (1,),
                      mode=lax.GatherScatterMode.PROMISE_IN_BOUNDS)


def _rsqrt_vec(x):
    i = lax.bitcast_convert_type(x, jnp.int32)
    i = jnp.int32(0x5F3759DF) - lax.shift_right_arithmetic(i, jnp.int32(1))
    y = lax.bitcast_convert_type(i, jnp.float32)
    for _ in range(2):
        y = y * (1.5 - 0.5 * x * y * y)
    return y


def _make_kernel(B, L, V):
    assert B % (NW * 128) == 0 and L % LC == 0
    ncell = (B // NW // SG) * (L // LC)
    ngrp_l = L // LC

    mesh = plsc.VectorSubcoreMesh(core_axis_name="c", subcore_axis_name="s")

    scratch = (
        [pltpu.VMEM((SG, LC, D), jnp.float32) for _ in range(2)]
        + [pltpu.VMEM((LO, 8, 8, SG + 1), jnp.float32) for _ in range(2)]
        + [pltpu.VMEM((SG, LC), jnp.int32) for _ in range(2)]
        + [pltpu.VMEM((SG, LC + 8), jnp.int32) for _ in range(2)]
        + [
            pltpu.VMEM((L, D), jnp.float32),
            pltpu.VMEM((2, D), jnp.float32),
            pltpu.VMEM((D,), jnp.float32),
            pltpu.VMEM((D,), jnp.float32),
            pltpu.VMEM((D,), jnp.float32),
        ]
        + [pltpu.SemaphoreType.DMA for _ in range(6)]
    )

    @functools.partial(
        pl.kernel,
        out_type=jax.ShapeDtypeStruct((L, D // 8, NW * 128 // 128, 8, 128),
                                      jnp.float32),
        mesh=mesh,
        compiler_params=pltpu.CompilerParams(
            needs_layout_passes=False, use_tc_tiling_on_sc=False),
        scratch_types=scratch,
    )
    def k(x_hbm, seg_hbm, tok_hbm, pos_hbm, segt_hbm, gam_hbm, bet_hbm,
          out_hbm, *refs):
        gbuf = refs[0:2]
        obuf = refs[2:4]
        xids = refs[4:6]
        sgids = refs[6:8]
        posb, segt_v, dd_v, gm_v, bt_v = refs[8:13]
        sems = refs[13:]
        sem_g = sems[0:2]
        sem_i = sems[2:4]
        sem_o = sems[4:6]

        wid = lax.axis_index("s") * NC + lax.axis_index("c")
        b00 = wid * 128

        pltpu.sync_copy(pos_hbm.at[pl.ds(0, L)], posb)
        pltpu.sync_copy(segt_hbm, segt_v)
        pltpu.sync_copy(gam_hbm, gm_v)
        pltpu.sync_copy(bet_hbm, bt_v)

        sl = [pl.ds(kk * LANES, LANES) for kk in range(ND)]

        def _fold_seg0(r, carry):
            for kk in range(ND):
                posb[r, sl[kk]] = posb[r, sl[kk]] + segt_v[0, sl[kk]]
            return carry
        lax.fori_loop(0, L, _fold_seg0, 0)
        for kk in range(ND):
            dd_v[sl[kk]] = segt_v[1, sl[kk]] - segt_v[0, sl[kk]]

        lanev = lax.iota(jnp.int32, LANES)
        dtv = [2 * kk + lax.shift_right_logical(lanev, jnp.int32(3))
               for kk in range(ND)]
        dsv = lanev & jnp.int32(7)

        def cell_gb(n):
            g = n // ngrp_l
            c = n % ngrp_l
            return b00 + g * SG, c * LC, g

        def stage(n, b, issue):
            b0, l0, _ = cell_gb(n)
            src_x = x_hbm.at[pl.ds(b0, SG), pl.ds(l0, LC)]
            src_s = seg_hbm.at[pl.ds(b0, SG), pl.ds(l0, LC)]
            dst_s = sgids[b].at[pl.ds(0, SG), pl.ds(0, LC)]
            if issue:
                pltpu.async_copy(src_x, xids[b], sem_i[b])
                pltpu.async_copy(src_s, dst_s, sem_i[b])
            else:
                pltpu.make_async_copy(src_x, xids[b], sem_i[b]).wait()
                pltpu.make_async_copy(src_s, dst_s, sem_i[b]).wait()

        def gathers(b, issue):
            for q in range(SG):
                src = tok_hbm.at[xids[b].at[q, :]]
                dst = gbuf[b].at[q]
                if issue:
                    pltpu.async_copy(src, dst, sem_g[b])
                else:
                    pltpu.make_async_copy(src, dst, sem_g[b]).wait()

        def out_dma(n, j, bo, issue):
            _, l0, g = cell_gb(n)
            dst = out_hbm.at[pl.ds(l0 + j * LO, LO), :, wid, :,
                             pl.ds(g * SG, SG)]
            src = obuf[bo].at[:, :, :, pl.ds(0, SG)]
            if issue:
                pass
            else:
                pass

        def _token(gb, ob, q, qv, lc, gl, li, sf16, dd, gm, bt):
            tok = [gb[q, lc, sl[kk]] for kk in range(ND)]
            pos = [posb[gl, sl[kk]] for kk in range(ND)]
            sb = _lane_bcast(sf16, li)
            e = [tok[kk] + pos[kk] + sb * dd[kk] for kk in range(ND)]
            tot = (e[0] + e[1]) + (e[2] + e[3])
            mean = _lane_bcast(plsc.cumsum(tot), LANES - 1) * (1.0 / D)
            c = [e[kk] - mean for kk in range(ND)]
            sq = (c[0] * c[0] + c[1] * c[1]) + (c[2] * c[2] + c[3] * c[3])
            var = _lane_bcast(plsc.cumsum(sq), LANES - 1) * (1.0 / D)
            rv = _rsqrt_vec(var + 1e-5)
            lqv = jnp.full((LANES,), li, jnp.int32)
            for kk in range(ND):
                val = c[kk] * (rv * gm[kk]) + bt[kk]
                plsc.store_scatter(ob, [lqv, dtv[kk], dsv, qv], val)

        def compute_tile(n, b, j, bo):
            _, l0, _ = cell_gb(n)
            dd = [dd_v[sl[kk]] for kk in range(ND)]
            gm = [gm_v[sl[kk]] for kk in range(ND)]
            bt = [bt_v[sl[kk]] for kk in range(ND)]

            def _qbody(q, carry):
                qv = jnp.full((LANES,), q, jnp.int32)
                sf16 = sgids[b][q, pl.ds(j * LO, 16)].astype(jnp.float32)
                for li in range(LO):
                    lc = j * LO + li
                    _token(gbuf[b], obuf[bo], q, qv, lc, l0 + lc, li,
                           sf16, dd, gm, bt)
                return carry
            lax.fori_loop(0, SG, _qbody, 0)

        pltpu.sync_copy(x_hbm.at[pl.ds(b00, SG), pl.ds(0, LC)], xids[0])
        pltpu.sync_copy(seg_hbm.at[pl.ds(b00, SG), pl.ds(0, LC)],
                        sgids[0].at[pl.ds(0, SG), pl.ds(0, LC)])
        stage(1, 1, True)
        gathers(0, True)

        def _iter(p, carry):
            for k2 in range(2):
                n = p * 2 + k2
                b = k2
                bn = 1 - k2
                gathers(b, False)
                if k2 == 0:
                    stage(n + 1, bn, False)
                    gathers(bn, True)
                else:
                    @pl.when(p <= ncell // 2 - 2)
                    def _g():
                        stage(n + 1, bn, False)
                        gathers(bn, True)
                for j in range(ngrp_l):
                    bo = (k2 + j) % 2
                    if k2 == 0 and j < 2:
                        @pl.when(p >= 1)
                        def _w(n=n, j=j, bo=bo):
                            out_dma(n, j, bo, False)
                    else:
                        out_dma(n, j, bo, False)
                    compute_tile(n, b, j, bo)
                    out_dma(n, j, bo, True)
                @pl.when(p <= ncell // 2 - 2)
                def _s():
                    stage(n + 2, b, True)
            return carry
        lax.fori_loop(0, ncell // 2, _iter, 0)

        out_dma(ncell - 1, ngrp_l - 2, (ncell - 1 + ngrp_l - 2) % 2, False)
        out_dma(ncell - 1, ngrp_l - 1, (ncell - 1 + ngrp_l - 1) % 2, False)

    return k


def kernel(x, seg, token_table, pos_table, seg_table, gamma, beta):
    B, L = x.shape
    V, d = token_table.shape
    k = _make_kernel(B, L, V)
    out5 = k(x.astype(jnp.int32), seg.astype(jnp.int32),
             token_table, pos_table, seg_table, gamma, beta)
    return out5.transpose(2, 4, 0, 1, 3).reshape(B, L, d)


# speedup vs baseline: 1.6839x; 1.4151x over previous
import functools

import jax
import jax.numpy as jnp
from jax import lax
from jax.experimental import pallas as pl
from jax.experimental.pallas import tpu as pltpu
from jax.experimental.pallas import tpu_sc as plsc

LANES = 16
NC = 2
NS = 16
NW = NC * NS

D = 64
ND = D // LANES
IB = 2
NBUF = 3

_GDN = lax.GatherDimensionNumbers(
    offset_dims=(), collapsed_slice_dims=(0,), start_index_map=(0,))


def _lane_bcast(v, lane):
    idx = jnp.full((LANES, 1), lane, dtype=jnp.int32)
    return lax.gather(v, idx, _GDN, (1,),
                      mode=lax.GatherScatterMode.PROMISE_IN_BOUNDS)


def _rsqrt_vec(x):
    i = lax.bitcast_convert_type(x, jnp.int32)
    i = jnp.int32(0x5F3759DF) - lax.shift_right_arithmetic(i, jnp.int32(1))
    y = lax.bitcast_convert_type(i, jnp.float32)
    for _ in range(2):
        y = y * (1.5 - 0.5 * x * y * y)
    return y


def _make_kernel(B, L, V):
    assert B % (NW * IB) == 0
    nblk = B // (NW * IB)
    s0, s1 = 104, L - 104
    ngrp = L // LANES
    tail = L - ngrp * LANES

    mesh = plsc.VectorSubcoreMesh(core_axis_name="c", subcore_axis_name="s")

    scratch = (
        [pltpu.VMEM((IB, L, D), jnp.float32) for _ in range(NBUF)]
        + [pltpu.VMEM((IB, L), jnp.int32) for _ in range(NBUF)]
        + [pltpu.VMEM((IB, L), jnp.int32) for _ in range(NBUF)]
        + [
            pltpu.VMEM((L, D), jnp.float32),
            pltpu.VMEM((2, D), jnp.float32),
            pltpu.VMEM((D,), jnp.float32),
            pltpu.VMEM((D,), jnp.float32),
            pltpu.VMEM((D,), jnp.float32),
        ]
        + [pltpu.SemaphoreType.DMA for _ in range(3 * NBUF)]
    )

    @functools.partial(
        pl.kernel,
        out_type=jax.ShapeDtypeStruct((B, L, D), jnp.float32),
        mesh=mesh,
        compiler_params=pltpu.CompilerParams(
            needs_layout_passes=False, use_tc_tiling_on_sc=False),
        scratch_types=scratch,
    )
    def k(x_hbm, seg_hbm, tok_hbm, pos_hbm, segt_hbm, gam_hbm, bet_hbm,
          out_hbm, *refs):
        rows = refs[0:NBUF]
        idxb = refs[NBUF:2 * NBUF]
        segb = refs[2 * NBUF:3 * NBUF]
        posb, segt_v, dd_v, gm_v, bt_v = refs[3 * NBUF:3 * NBUF + 5]
        sems = refs[3 * NBUF + 5:]
        sem_g = sems[0:NBUF]
        sem_i = sems[NBUF:2 * NBUF]
        sem_o = sems[2 * NBUF:]

        wid = lax.axis_index("s") * NC + lax.axis_index("c")
        seq00 = wid * (nblk * IB)

        pltpu.sync_copy(pos_hbm.at[pl.ds(0, L)], posb)
        pltpu.sync_copy(segt_hbm, segt_v)
        pltpu.sync_copy(gam_hbm, gm_v)
        pltpu.sync_copy(bet_hbm, bt_v)

        sl = [pl.ds(kk * LANES, LANES) for kk in range(ND)]

        def _fold_seg0(r, carry):
            for kk in range(ND):
                posb[r, sl[kk]] = posb[r, sl[kk]] + segt_v[0, sl[kk]]
            return carry
        lax.fori_loop(0, L, _fold_seg0, 0)
        for kk in range(ND):
            dd_v[sl[kk]] = segt_v[1, sl[kk]] - segt_v[0, sl[kk]]

        def seqbase(m):
            return seq00 + m * IB

        def stage(m, b, issue):
            src_x = x_hbm.at[pl.ds(seqbase(m), IB)]
            src_s = seg_hbm.at[pl.ds(seqbase(m), IB)]
            if issue:
                pltpu.async_copy(src_x, idxb[b], sem_i[b])
                pltpu.async_copy(src_s, segb[b], sem_i[b])
            else:
                pltpu.make_async_copy(src_x, idxb[b], sem_i[b]).wait()
                pltpu.make_async_copy(src_s, segb[b], sem_i[b]).wait()

        def gathers(b, issue):
            for q in range(IB):
                for (off, n) in ((0, s0), (s0, s1)):
                    src = tok_hbm.at[idxb[b].at[q, pl.ds(off, n)]]
                    dst = rows[b].at[q, pl.ds(off, n), :]
                    if issue:
                        pltpu.async_copy(src, dst, sem_g[b])
                    else:
                        pltpu.make_async_copy(src, dst, sem_g[b]).wait()

        def out_dma(m, b, issue):
            dst = out_hbm.at[pl.ds(seqbase(m), IB)]
            if issue:
                pltpu.async_copy(rows[b], dst, sem_o[b])
            else:
                pltpu.make_async_copy(rows[b], dst, sem_o[b]).wait()

        def _token(rv, q, t, i, sf, dd, gm, bt):
            tok = [rv[q, t, sl[kk]] for kk in range(ND)]
            pos = [posb[t, sl[kk]] for kk in range(ND)]
            sb = _lane_bcast(sf, i)
            e = [tok[kk] + pos[kk] + sb * dd[kk] for kk in range(ND)]
            tot = (e[0] + e[1]) + (e[2] + e[3])
            mean = _lane_bcast(plsc.cumsum(tot), LANES - 1) * (1.0 / D)
            c = [e[kk] - mean for kk in range(ND)]
            sq = (c[0] * c[0] + c[1] * c[1]) + (c[2] * c[2] + c[3] * c[3])
            var = _lane_bcast(plsc.cumsum(sq), LANES - 1) * (1.0 / D)
            rv_ = _rsqrt_vec(var + 1e-5)
            for kk in range(ND):
                rv[q, t, sl[kk]] = c[kk] * (rv_ * gm[kk]) + bt[kk]

        def compute(b):
            dd = [dd_v[sl[kk]] for kk in range(ND)]
            gm = [gm_v[sl[kk]] for kk in range(ND)]
            bt = [bt_v[sl[kk]] for kk in range(ND)]
            for q in range(IB):
                def _grp(g, carry, q=q):
                    base = g * LANES
                    sf = segb[b][q, pl.ds(base, LANES)].astype(jnp.float32)
                    for i in range(LANES):
                        _token(rows[b], q, base + i, i, sf, dd, gm, bt)
                    return carry
                lax.fori_loop(0, ngrp, _grp, 0)

                base = L - LANES
                sf = segb[b][q, pl.ds(base, LANES)].astype(jnp.float32)
                for i in range(LANES - tail, LANES):
                    _token(rows[b], q, base + i, i, sf, dd, gm, bt)

            pass

        pltpu.sync_copy(x_hbm.at[pl.ds(seqbase(0), IB)], idxb[0])
        pltpu.sync_copy(seg_hbm.at[pl.ds(seqbase(0), IB)], segb[0])
        stage(1, 1, True)
        gathers(0, True)

        def _iter(p, carry):
            for kk3 in range(NBUF):
                m = p * NBUF + kk3
                b = kk3
                bn = (kk3 + 1) % NBUF
                bs = (kk3 + 2) % NBUF
                gathers(b, False)
                stage(m + 1, bn, False)
                if kk3 == 2:
                    out_dma(m - 2, bn, False)
                else:
                    @pl.when(p >= 1)
                    def _w():
                        out_dma(m - 2, bn, False)
                gathers(bn, True)
                if kk3 == 2:
                    @pl.when(p <= (nblk - 5) // NBUF)
                    def _s():
                        stage(m + 2, bs, True)
                else:
                    stage(m + 2, bs, True)
                compute(b)
                out_dma(m, b, True)
            return carry
        lax.fori_loop(0, (nblk - 1) // NBUF, _iter, 0)

        mlast = nblk - 1
        gathers(0, False)
        compute(0)
        out_dma(mlast, 0, True)
        out_dma(mlast - 2, 1, False)
        out_dma(mlast - 1, 2, False)
        out_dma(mlast, 0, False)

    return k


def kernel(x, seg, token_table, pos_table, seg_table, gamma, beta):
    B, L = x.shape
    V, d = token_table.shape
    k = _make_kernel(B, L, V)
    return k(x.astype(jnp.int32), seg.astype(jnp.int32),
             token_table, pos_table, seg_table, gamma, beta)


# speedup vs baseline: 1.6859x; 1.0012x over previous
import functools

import jax
import jax.numpy as jnp
from jax import lax
from jax.experimental import pallas as pl
from jax.experimental.pallas import tpu as pltpu
from jax.experimental.pallas import tpu_sc as plsc

LANES = 16
NC = 2
NS = 16
NW = NC * NS

D = 64
ND = D // LANES
IB = 2
NBUF = 3

_GDN = lax.GatherDimensionNumbers(
    offset_dims=(), collapsed_slice_dims=(0,), start_index_map=(0,))


def _lane_bcast(v, lane):
    idx = jnp.full((LANES, 1), lane, dtype=jnp.int32)
    return lax.gather(v, idx, _GDN, (1,),
                      mode=lax.GatherScatterMode.PROMISE_IN_BOUNDS)


def _rsqrt_vec(x):
    i = lax.bitcast_convert_type(x, jnp.int32)
    i = jnp.int32(0x5F3759DF) - lax.shift_right_arithmetic(i, jnp.int32(1))
    y = lax.bitcast_convert_type(i, jnp.float32)
    for _ in range(2):
        y = y * (1.5 - 0.5 * x * y * y)
    return y


def _make_kernel(B, L, V):
    assert B % (NW * IB) == 0
    nblk = B // (NW * IB)
    s0, s1 = 104, L - 104
    ngrp = L // LANES
    tail = L - ngrp * LANES

    mesh = plsc.VectorSubcoreMesh(core_axis_name="c", subcore_axis_name="s")

    scratch = (
        [pltpu.VMEM((IB, L, D), jnp.float32) for _ in range(NBUF)]
        + [pltpu.VMEM((IB, L), jnp.int32) for _ in range(NBUF)]
        + [pltpu.VMEM((IB, L), jnp.int32) for _ in range(NBUF)]
        + [
            pltpu.VMEM((L, D), jnp.float32),
            pltpu.VMEM((2, D), jnp.float32),
            pltpu.VMEM((D,), jnp.float32),
            pltpu.VMEM((D,), jnp.float32),
            pltpu.VMEM((D,), jnp.float32),
        ]
        + [pltpu.SemaphoreType.DMA for _ in range(3 * NBUF)]
    )

    @functools.partial(
        pl.kernel,
        out_type=jax.ShapeDtypeStruct((B, L, D), jnp.float32),
        mesh=mesh,
        compiler_params=pltpu.CompilerParams(
            needs_layout_passes=False, use_tc_tiling_on_sc=False),
        scratch_types=scratch,
    )
    def k(x_hbm, seg_hbm, tok_hbm, pos_hbm, segt_hbm, gam_hbm, bet_hbm,
          out_hbm, *refs):
        rows = refs[0:NBUF]
        idxb = refs[NBUF:2 * NBUF]
        segb = refs[2 * NBUF:3 * NBUF]
        posb, segt_v, dd_v, gm_v, bt_v = refs[3 * NBUF:3 * NBUF + 5]
        sems = refs[3 * NBUF + 5:]
        sem_g = sems[0:NBUF]
        sem_i = sems[NBUF:2 * NBUF]
        sem_o = sems[2 * NBUF:]

        wid = lax.axis_index("s") * NC + lax.axis_index("c")
        seq00 = wid * (nblk * IB)

        pltpu.sync_copy(pos_hbm.at[pl.ds(0, L)], posb)
        pltpu.sync_copy(segt_hbm, segt_v)
        pltpu.sync_copy(gam_hbm, gm_v)
        pltpu.sync_copy(bet_hbm, bt_v)

        sl = [pl.ds(kk * LANES, LANES) for kk in range(ND)]

        def _fold_seg0(r, carry):
            for kk in range(ND):
                posb[r, sl[kk]] = posb[r, sl[kk]] + segt_v[0, sl[kk]]
            return carry
        lax.fori_loop(0, L, _fold_seg0, 0)
        for kk in range(ND):
            dd_v[sl[kk]] = segt_v[1, sl[kk]] - segt_v[0, sl[kk]]

        def seqbase(m):
            return seq00 + m * IB

        def stage(m, b, issue):
            src_x = x_hbm.at[pl.ds(seqbase(m), IB)]
            src_s = seg_hbm.at[pl.ds(seqbase(m), IB)]
            if issue:
                pltpu.async_copy(src_x, idxb[b], sem_i[b])
                pltpu.async_copy(src_s, segb[b], sem_i[b])
            else:
                pltpu.make_async_copy(src_x, idxb[b], sem_i[b]).wait()
                pltpu.make_async_copy(src_s, segb[b], sem_i[b]).wait()

        def gathers(b, issue):
            for q in range(IB):
                for (off, n) in ((0, s0), (s0, s1)):
                    src = tok_hbm.at[idxb[b].at[q, pl.ds(off, n)]]
                    dst = rows[b].at[q, pl.ds(off, n), :]
                    if issue:
                        pltpu.async_copy(src, dst, sem_g[b])
                    else:
                        pltpu.make_async_copy(src, dst, sem_g[b]).wait()

        def out_dma(m, b, issue):
            dst = out_hbm.at[pl.ds(seqbase(m), IB)]
            if issue:
                pass
            else:
                pass

        def _token(rv, q, t, i, sf, dd, gm, bt):
            tok = [rv[q, t, sl[kk]] for kk in range(ND)]
            pos = [posb[t, sl[kk]] for kk in range(ND)]
            sb = _lane_bcast(sf, i)
            e = [tok[kk] + pos[kk] + sb * dd[kk] for kk in range(ND)]
            tot = (e[0] + e[1]) + (e[2] + e[3])
            mean = _lane_bcast(plsc.cumsum(tot), LANES - 1) * (1.0 / D)
            c = [e[kk] - mean for kk in range(ND)]
            sq = (c[0] * c[0] + c[1] * c[1]) + (c[2] * c[2] + c[3] * c[3])
            var = _lane_bcast(plsc.cumsum(sq), LANES - 1) * (1.0 / D)
            rv_ = _rsqrt_vec(var + 1e-5)
            for kk in range(ND):
                rv[q, t, sl[kk]] = c[kk] * (rv_ * gm[kk]) + bt[kk]

        def compute(b):
            dd = [dd_v[sl[kk]] for kk in range(ND)]
            gm = [gm_v[sl[kk]] for kk in range(ND)]
            bt = [bt_v[sl[kk]] for kk in range(ND)]
            for q in range(IB):
                def _grp(g, carry, q=q):
                    base = g * LANES
                    sf = segb[b][q, pl.ds(base, LANES)].astype(jnp.float32)
                    for i in range(LANES):
                        _token(rows[b], q, base + i, i, sf, dd, gm, bt)
                    return carry
                lax.fori_loop(0, ngrp, _grp, 0)

                base = L - LANES
                sf = segb[b][q, pl.ds(base, LANES)].astype(jnp.float32)
                for i in range(LANES - tail, LANES):
                    _token(rows[b], q, base + i, i, sf, dd, gm, bt)

        pltpu.sync_copy(x_hbm.at[pl.ds(seqbase(0), IB)], idxb[0])
        pltpu.sync_copy(seg_hbm.at[pl.ds(seqbase(0), IB)], segb[0])
        stage(1, 1, True)
        gathers(0, True)

        def _iter(p, carry):
            for kk3 in range(NBUF):
                m = p * NBUF + kk3
                b = kk3
                bn = (kk3 + 1) % NBUF
                bs = (kk3 + 2) % NBUF
                gathers(b, False)
                stage(m + 1, bn, False)
                if kk3 == 2:
                    out_dma(m - 2, bn, False)
                else:
                    @pl.when(p >= 1)
                    def _w():
                        out_dma(m - 2, bn, False)
                gathers(bn, True)
                if kk3 == 2:
                    @pl.when(p <= (nblk - 5) // NBUF)
                    def _s():
                        stage(m + 2, bs, True)
                else:
                    stage(m + 2, bs, True)
                compute(b)
                out_dma(m, b, True)
            return carry
        lax.fori_loop(0, (nblk - 1) // NBUF, _iter, 0)

        mlast = nblk - 1
        gathers(0, False)
        compute(0)
        out_dma(mlast, 0, True)
        out_dma(mlast - 2, 1, False)
        out_dma(mlast - 1, 2, False)
        out_dma(mlast, 0, False)

    return k


def kernel(x, seg, token_table, pos_table, seg_table, gamma, beta):
    B, L = x.shape
    V, d = token_table.shape
    k = _make_kernel(B, L, V)
    return k(x.astype(jnp.int32), seg.astype(jnp.int32),
             token_table, pos_table, seg_table, gamma, beta)


# speedup vs baseline: 2.0463x; 1.2138x over previous
import functools

import jax
import jax.numpy as jnp
from jax import lax
from jax.experimental import pallas as pl
from jax.experimental.pallas import tpu as pltpu
from jax.experimental.pallas import tpu_sc as plsc

LANES = 16
NC = 2
NS = 16
NW = NC * NS

D = 64
ND = D // LANES
IB = 2
NBUF = 3

_GDN = lax.GatherDimensionNumbers(
    offset_dims=(), collapsed_slice_dims=(0,), start_index_map=(0,))


def _lane_bcast(v, lane):
    idx = jnp.full((LANES, 1), lane, dtype=jnp.int32)
    return lax.gather(v, idx, _GDN, (1,),
                      mode=lax.GatherScatterMode.PROMISE_IN_BOUNDS)


def _rsqrt_vec(x):
    i = lax.bitcast_convert_type(x, jnp.int32)
    i = jnp.int32(0x5F3759DF) - lax.shift_right_arithmetic(i, jnp.int32(1))
    y = lax.bitcast_convert_type(i, jnp.float32)
    for _ in range(1):
        y = y * (1.5 - 0.5 * x * y * y)
    return y


def _make_kernel(B, L, V):
    assert B % (NW * IB) == 0
    nblk = B // (NW * IB)
    s0, s1 = 104, L - 104
    ngrp = L // LANES
    tail = L - ngrp * LANES

    mesh = plsc.VectorSubcoreMesh(core_axis_name="c", subcore_axis_name="s")

    scratch = (
        [pltpu.VMEM((IB, L, D), jnp.float32) for _ in range(NBUF)]
        + [pltpu.VMEM((IB, L), jnp.int32) for _ in range(NBUF)]
        + [pltpu.VMEM((IB, L), jnp.int32) for _ in range(NBUF)]
        + [
            pltpu.VMEM((L, D), jnp.float32),
            pltpu.VMEM((2, D), jnp.float32),
            pltpu.VMEM((D,), jnp.float32),
            pltpu.VMEM((D,), jnp.float32),
            pltpu.VMEM((D,), jnp.float32),
        ]
        + [pltpu.SemaphoreType.DMA for _ in range(3 * NBUF)]
    )

    @functools.partial(
        pl.kernel,
        out_type=jax.ShapeDtypeStruct((B, L, D), jnp.float32),
        mesh=mesh,
        compiler_params=pltpu.CompilerParams(
            needs_layout_passes=False, use_tc_tiling_on_sc=False),
        scratch_types=scratch,
    )
    def k(x_hbm, seg_hbm, tok_hbm, pos_hbm, segt_hbm, gam_hbm, bet_hbm,
          out_hbm, *refs):
        rows = refs[0:NBUF]
        idxb = refs[NBUF:2 * NBUF]
        segb = refs[2 * NBUF:3 * NBUF]
        posb, segt_v, dd_v, gm_v, bt_v = refs[3 * NBUF:3 * NBUF + 5]
        sems = refs[3 * NBUF + 5:]
        sem_g = sems[0:NBUF]
        sem_i = sems[NBUF:2 * NBUF]
        sem_o = sems[2 * NBUF:]

        wid = lax.axis_index("s") * NC + lax.axis_index("c")
        seq00 = wid * (nblk * IB)

        pltpu.sync_copy(pos_hbm.at[pl.ds(0, L)], posb)
        pltpu.sync_copy(segt_hbm, segt_v)
        pltpu.sync_copy(gam_hbm, gm_v)
        pltpu.sync_copy(bet_hbm, bt_v)

        sl = [pl.ds(kk * LANES, LANES) for kk in range(ND)]

        def _fold_seg0(r, carry):
            for kk in range(ND):
                posb[r, sl[kk]] = posb[r, sl[kk]] + segt_v[0, sl[kk]]
            return carry
        lax.fori_loop(0, L, _fold_seg0, 0)
        for kk in range(ND):
            dd_v[sl[kk]] = segt_v[1, sl[kk]] - segt_v[0, sl[kk]]

        def seqbase(m):
            return seq00 + m * IB

        def stage(m, b, issue):
            src_x = x_hbm.at[pl.ds(seqbase(m), IB)]
            src_s = seg_hbm.at[pl.ds(seqbase(m), IB)]
            if issue:
                pltpu.async_copy(src_x, idxb[b], sem_i[b])
                pltpu.async_copy(src_s, segb[b], sem_i[b])
            else:
                pltpu.make_async_copy(src_x, idxb[b], sem_i[b]).wait()
                pltpu.make_async_copy(src_s, segb[b], sem_i[b]).wait()

        def gathers(b, issue):
            for q in range(IB):
                for (off, n) in ((0, s0), (s0, s1)):
                    src = tok_hbm.at[idxb[b].at[q, pl.ds(off, n)]]
                    dst = rows[b].at[q, pl.ds(off, n), :]
                    if issue:
                        pltpu.async_copy(src, dst, sem_g[b])
                    else:
                        pltpu.make_async_copy(src, dst, sem_g[b]).wait()

        def out_dma(m, b, issue):
            dst = out_hbm.at[pl.ds(seqbase(m), IB)]
            if issue:
                pltpu.async_copy(rows[b], dst, sem_o[b])
            else:
                pltpu.make_async_copy(rows[b], dst, sem_o[b]).wait()

        def _token(rv, q, t, i, sf, dd, gm, bt):
            tok = [rv[q, t, sl[kk]] for kk in range(ND)]
            pos = [posb[t, sl[kk]] for kk in range(ND)]
            sb = _lane_bcast(sf, i)
            e = [tok[kk] + pos[kk] + sb * dd[kk] for kk in range(ND)]
            tot = (e[0] + e[1]) + (e[2] + e[3])
            sq = (e[0] * e[0] + e[1] * e[1]) + (e[2] * e[2] + e[3] * e[3])
            mean = _lane_bcast(plsc.cumsum(tot), LANES - 1) * (1.0 / D)
            m2 = _lane_bcast(plsc.cumsum(sq), LANES - 1) * (1.0 / D)
            var = m2 - mean * mean
            c = [e[kk] - mean for kk in range(ND)]
            rv_ = _rsqrt_vec(var + 1e-5)
            for kk in range(ND):
                rv[q, t, sl[kk]] = c[kk] * (rv_ * gm[kk]) + bt[kk]

        def compute(b):
            dd = [dd_v[sl[kk]] for kk in range(ND)]
            gm = [gm_v[sl[kk]] for kk in range(ND)]
            bt = [bt_v[sl[kk]] for kk in range(ND)]
            for q in range(IB):
                def _grp(g, carry, q=q):
                    base = g * LANES
                    sf = segb[b][q, pl.ds(base, LANES)].astype(jnp.float32)
                    for i in range(LANES):
                        _token(rows[b], q, base + i, i, sf, dd, gm, bt)
                    return carry
                lax.fori_loop(0, ngrp, _grp, 0)

                base = L - LANES
                sf = segb[b][q, pl.ds(base, LANES)].astype(jnp.float32)
                for i in range(LANES - tail, LANES):
                    _token(rows[b], q, base + i, i, sf, dd, gm, bt)

        pltpu.sync_copy(x_hbm.at[pl.ds(seqbase(0), IB)], idxb[0])
        pltpu.sync_copy(seg_hbm.at[pl.ds(seqbase(0), IB)], segb[0])
        stage(1, 1, True)
        gathers(0, True)

        def _iter(p, carry):
            for kk3 in range(NBUF):
                m = p * NBUF + kk3
                b = kk3
                bn = (kk3 + 1) % NBUF
                bs = (kk3 + 2) % NBUF
                gathers(b, False)
                stage(m + 1, bn, False)
                if kk3 == 2:
                    out_dma(m - 2, bn, False)
                else:
                    @pl.when(p >= 1)
                    def _w():
                        out_dma(m - 2, bn, False)
                gathers(bn, True)
                if kk3 == 2:
                    @pl.when(p <= (nblk - 5) // NBUF)
                    def _s():
                        stage(m + 2, bs, True)
                else:
                    stage(m + 2, bs, True)
                compute(b)
                out_dma(m, b, True)
            return carry
        lax.fori_loop(0, (nblk - 1) // NBUF, _iter, 0)

        mlast = nblk - 1
        gathers(0, False)
        compute(0)
        out_dma(mlast, 0, True)
        out_dma(mlast - 2, 1, False)
        out_dma(mlast - 1, 2, False)
        out_dma(mlast, 0, False)

    return k


def kernel(x, seg, token_table, pos_table, seg_table, gamma, beta):
    B, L = x.shape
    V, d = token_table.shape
    k = _make_kernel(B, L, V)
    return k(x.astype(jnp.int32), seg.astype(jnp.int32),
             token_table, pos_table, seg_table, gamma, beta)


# speedup vs baseline: 2.0601x; 1.0067x over previous
import functools

import jax
import jax.numpy as jnp
from jax import lax
from jax.experimental import pallas as pl
from jax.experimental.pallas import tpu as pltpu
from jax.experimental.pallas import tpu_sc as plsc

LANES = 16
NC = 2
NS = 16
NW = NC * NS

D = 64
ND = D // LANES
IB = 2

_GDN = lax.GatherDimensionNumbers(
    offset_dims=(), collapsed_slice_dims=(0,), start_index_map=(0,))


def _lane_bcast(v, lane):
    idx = jnp.full((LANES, 1), lane, dtype=jnp.int32)
    return lax.gather(v, idx, _GDN, (1,),
                      mode=lax.GatherScatterMode.PROMISE_IN_BOUNDS)


def _rsqrt_vec(x):
    i = lax.bitcast_convert_type(x, jnp.int32)
    i = jnp.int32(0x5F3759DF) - lax.shift_right_arithmetic(i, jnp.int32(1))
    y = lax.bitcast_convert_type(i, jnp.float32)
    for _ in range(1):
        y = y * (1.5 - 0.5 * x * y * y)
    return y


def _make_kernel(B, L, V):
    assert B % (NW * IB) == 0
    nblk = B // (NW * IB)
    s0, s1 = 104, L - 104
    ngrp = L // LANES
    tail = L - ngrp * LANES
    orow = IB * L * D // 128

    mesh = plsc.VectorSubcoreMesh(core_axis_name="c", subcore_axis_name="s")

    scratch = (
        [pltpu.VMEM((IB, L, D), jnp.float32) for _ in range(2)]
        + [pltpu.VMEM((IB, L // 8, 8, D), jnp.float32) for _ in range(2)]
        + [pltpu.VMEM((IB, L), jnp.int32) for _ in range(2)]
        + [pltpu.VMEM((IB, L), jnp.int32) for _ in range(2)]
        + [
            pltpu.VMEM((L, D), jnp.float32),
            pltpu.VMEM((2, D), jnp.float32),
            pltpu.VMEM((D,), jnp.float32),
            pltpu.VMEM((D,), jnp.float32),
            pltpu.VMEM((D,), jnp.float32),
        ]
        + [pltpu.SemaphoreType.DMA for _ in range(6)]
    )

    @functools.partial(
        pl.kernel,
        out_type=jax.ShapeDtypeStruct((B, L // 8, 8, 128), jnp.float32),
        mesh=mesh,
        compiler_params=pltpu.CompilerParams(
            needs_layout_passes=False, use_tc_tiling_on_sc=False),
        scratch_types=scratch,
    )
    def k(x_hbm, seg_hbm, tok_hbm, pos_hbm, segt_hbm, gam_hbm, bet_hbm,
          out_hbm, *refs):
        gbuf = refs[0:2]
        obuf = refs[2:4]
        idxb = refs[4:6]
        segb = refs[6:8]
        posb, segt_v, dd_v, gm_v, bt_v = refs[8:13]
        sems = refs[13:]
        sem_g = sems[0:2]
        sem_i = sems[2:4]
        sem_o = sems[4:6]

        wid = lax.axis_index("s") * NC + lax.axis_index("c")
        seq00 = wid * (nblk * IB)

        pltpu.sync_copy(pos_hbm.at[pl.ds(0, L)], posb)
        pltpu.sync_copy(segt_hbm, segt_v)
        pltpu.sync_copy(gam_hbm, gm_v)
        pltpu.sync_copy(bet_hbm, bt_v)

        sl = [pl.ds(kk * LANES, LANES) for kk in range(ND)]

        def _fold_seg0(r, carry):
            for kk in range(ND):
                posb[r, sl[kk]] = posb[r, sl[kk]] + segt_v[0, sl[kk]]
            return carry
        lax.fori_loop(0, L, _fold_seg0, 0)
        for kk in range(ND):
            dd_v[sl[kk]] = segt_v[1, sl[kk]] - segt_v[0, sl[kk]]

        def seqbase(m):
            return seq00 + m * IB

        def stage(m, b, issue):
            src_x = x_hbm.at[pl.ds(seqbase(m), IB)]
            src_s = seg_hbm.at[pl.ds(seqbase(m), IB)]
            if issue:
                pltpu.async_copy(src_x, idxb[b], sem_i[b])
                pltpu.async_copy(src_s, segb[b], sem_i[b])
            else:
                pltpu.make_async_copy(src_x, idxb[b], sem_i[b]).wait()
                pltpu.make_async_copy(src_s, segb[b], sem_i[b]).wait()

        def gathers(b, issue):
            for q in range(IB):
                for (off, n) in ((0, s0), (s0, s1)):
                    src = tok_hbm.at[idxb[b].at[q, pl.ds(off, n)]]
                    dst = gbuf[b].at[q, pl.ds(off, n), :]
                    if issue:
                        pltpu.async_copy(src, dst, sem_g[b])
                    else:
                        pltpu.make_async_copy(src, dst, sem_g[b]).wait()

        def out_dma(m, b, issue):
            dst = out_hbm.at[pl.ds(seqbase(m), IB), pl.ds(0, L // 8),
                             pl.ds(0, 8), pl.ds(0, D)]
            if issue:
                pltpu.async_copy(obuf[b], dst, sem_o[b])
            else:
                pltpu.make_async_copy(obuf[b], dst, sem_o[b]).wait()

        def _token(gb, ob, q, t, orow_base, i, sf, dd, gm, bt):
            tok = [gb[q, t, sl[kk]] for kk in range(ND)]
            pos = [posb[t, sl[kk]] for kk in range(ND)]
            sb = _lane_bcast(sf, i)
            e = [tok[kk] + pos[kk] + sb * dd[kk] for kk in range(ND)]
            tot = (e[0] + e[1]) + (e[2] + e[3])
            sq = (e[0] * e[0] + e[1] * e[1]) + (e[2] * e[2] + e[3] * e[3])
            mean = _lane_bcast(plsc.cumsum(tot), LANES - 1) * (1.0 / D)
            m2 = _lane_bcast(plsc.cumsum(sq), LANES - 1) * (1.0 / D)
            var = m2 - mean * mean
            c = [e[kk] - mean for kk in range(ND)]
            rv = _rsqrt_vec(var + 1e-5)
            r = orow_base + i // 8
            c8 = i % 8
            for kk in range(ND):
                ob[q, r, c8, sl[kk]] = c[kk] * (rv * gm[kk]) + bt[kk]

        def compute(b):
            dd = [dd_v[sl[kk]] for kk in range(ND)]
            gm = [gm_v[sl[kk]] for kk in range(ND)]
            bt = [bt_v[sl[kk]] for kk in range(ND)]
            for q in range(IB):
                def _grp(g, carry, q=q):
                    base = g * LANES
                    sf = segb[b][q, pl.ds(base, LANES)].astype(jnp.float32)
                    ob_base = g * (LANES // 8)
                    for i in range(LANES):
                        _token(gbuf[b], obuf[b], q, base + i, ob_base, i,
                               sf, dd, gm, bt)
                    return carry
                lax.fori_loop(0, ngrp, _grp, 0)
                base = L - LANES
                sf = segb[b][q, pl.ds(base, LANES)].astype(jnp.float32)
                ob_base = base // 8
                for i in range(LANES - tail, LANES):
                    _token(gbuf[b], obuf[b], q, base + i, ob_base, i,
                           sf, dd, gm, bt)

        pltpu.sync_copy(x_hbm.at[pl.ds(seqbase(0), IB)], idxb[0])
        pltpu.sync_copy(seg_hbm.at[pl.ds(seqbase(0), IB)], segb[0])
        stage(1, 1, True)
        gathers(0, True)

        def _iter(p, carry):
            for k2 in range(2):
                m = p * 2 + k2
                b = k2
                bn = 1 - k2
                gathers(b, False)
                if k2 == 0:
                    stage(m + 1, bn, False)
                    gathers(bn, True)
                else:
                    @pl.when(p <= nblk // 2 - 2)
                    def _g():
                        stage(m + 1, bn, False)
                        gathers(bn, True)
                @pl.when(p >= 1)
                def _w():
                    out_dma(m - 2, b, False)
                compute(b)
                out_dma(m, b, True)
                @pl.when(p <= nblk // 2 - 2)
                def _s():
                    stage(m + 2, b, True)
            return carry
        lax.fori_loop(0, nblk // 2, _iter, 0)

        out_dma(nblk - 2, 0, False)
        out_dma(nblk - 1, 1, False)

    return k


def kernel(x, seg, token_table, pos_table, seg_table, gamma, beta):
    B, L = x.shape
    V, d = token_table.shape
    k = _make_kernel(B, L, V)
    out4 = k(x.astype(jnp.int32), seg.astype(jnp.int32),
             token_table, pos_table, seg_table, gamma, beta)
    return out4[:, :, :, :d].reshape(B, L, d)
